# baseline (speedup 1.0000x reference)
import functools

import jax
import jax.numpy as jnp
from jax import lax
from jax.experimental import pallas as pl
from jax.experimental.pallas import tpu as pltpu
from jax.experimental.pallas import tpu_sc as plsc

VOCAB = 1000000
EMBED = 64
BATCH = 4096
MAXLEN = 200

NC = 2
NS = 16
NW = NC * NS
TOTAL = BATCH * MAXLEN
BPW = TOTAL // NW
CH = 128
NCHUNK = BPW // CH
NBUF = 8
NSTEP = NCHUNK // NBUF

_mesh = plsc.VectorSubcoreMesh(core_axis_name="c", subcore_axis_name="s")


@functools.partial(
    pl.kernel,
    mesh=_mesh,
    compiler_params=pltpu.CompilerParams(use_tc_tiling_on_sc=False),
    out_type=jax.ShapeDtypeStruct((TOTAL, EMBED), jnp.float32),
    scratch_types=[pltpu.VMEM((NCHUNK, CH), jnp.int32),
                   pltpu.VMEM((NBUF, CH, EMBED), jnp.float32)]
    + [pltpu.SemaphoreType.DMA] * (2 * NBUF),
)
def _emb_lookup(idx_hbm, table_hbm, out_hbm, idx_v, rows_v, *sems):
    gsem = sems[:NBUF]
    wsem = sems[NBUF:]
    wid = lax.axis_index("s") * NC + lax.axis_index("c")
    base = wid * BPW

    pltpu.sync_copy(idx_hbm.at[pl.ds(wid * NCHUNK, NCHUNK)], idx_v)

    def gather(c, b):
        return pltpu.make_async_copy(
            table_hbm.at[idx_v.at[c]], rows_v.at[b], gsem[b])

    def write(c, b):
        return pltpu.make_async_copy(
            rows_v.at[b], out_hbm.at[pl.ds(base + c * CH, CH)], wsem[b])

    for b in range(NBUF):
        gather(b, b).start()

    def body(s, carry):
        for b in range(NBUF):
            c = s * NBUF + b
            gather(c, b).wait()
            write(c, b).start()
        for b in range(NBUF):
            c = s * NBUF + b
            write(c, b).wait()
            gather(c + NBUF, b).start()
        return carry

    lax.fori_loop(0, NSTEP - 1, body, 0)

    s = NSTEP - 1
    for b in range(NBUF):
        c = s * NBUF + b
        gather(c, b).wait()
        write(c, b).start()
    for b in range(NBUF):
        write(s * NBUF + b, b).wait()


def kernel(inputs, table):
    idx = inputs.reshape(NW * NCHUNK, CH).astype(jnp.int32)
    out = _emb_lookup(idx, table)
    return out.reshape(BATCH, MAXLEN, EMBED)
